# baseline (speedup 1.0000x reference)
import functools

import jax
import jax.numpy as jnp
from jax import lax
from jax.experimental import pallas as pl
from jax.experimental.pallas import tpu as pltpu
from jax.experimental.pallas import tpu_sc as plsc

VOCAB = 100000
HIDDEN = 128
SEQ = 2048
BATCH = 4
EPS = 1e-12
L = 16
NC = 2
NS = 16
NW = NC * NS
NTOK = BATCH * SEQ
TPW = NTOK // NW
WPB = SEQ // TPW
GROUPS = TPW // L
CHUNKS = HIDDEN // L
NQ = 4
QROWS = TPW // NQ
QGROUPS = GROUPS // NQ


_DN = lax.GatherDimensionNumbers(
    offset_dims=(), collapsed_slice_dims=(0,), start_index_map=(0,))


def _gather16(v, idx):
    return lax.gather(v, idx[:, None], _DN, slice_sizes=(1,),
                      mode=lax.GatherScatterMode.PROMISE_IN_BOUNDS)


def _lane_bcast(v, j):
    return _gather16(v, jnp.full((L,), j, dtype=jnp.int32))


_MESH = plsc.VectorSubcoreMesh(core_axis_name="c", subcore_axis_name="s")


@functools.partial(
    pl.kernel,
    out_type=jax.ShapeDtypeStruct((NTOK, HIDDEN), jnp.float32),
    mesh=_MESH,
    scratch_types=[
        pltpu.VMEM((TPW,), jnp.int32),
        pltpu.VMEM((TPW, HIDDEN), jnp.float32),
        pltpu.VMEM((TPW, HIDDEN), jnp.float32),
        pltpu.VMEM((TPW,), jnp.int32),
        pltpu.VMEM((TPW,), jnp.int32),
        pltpu.VMEM((2, HIDDEN), jnp.float32),
        pltpu.VMEM((2, HIDDEN), jnp.float32),
        pltpu.SemaphoreType.DMA,
        pltpu.SemaphoreType.DMA,
        pltpu.SemaphoreType.DMA,
        pltpu.SemaphoreType.DMA,
        pltpu.SemaphoreType.DMA,
    ],
)
def _emb_sum_kernel(ids_hbm, tt_hbm, wm_hbm, word_hbm, pos_hbm, ty_hbm,
                    ar_hbm, out_hbm,
                    idx_v, w_v, p_v, tt_v, wm_v, ty_v, ar_v,
                    semA, semQ1, semQ2, semQ3, semC):
    cid = lax.axis_index("c")
    sid = lax.axis_index("s")
    wid = sid * NC + cid
    b = wid // WPB
    col0 = (wid % WPB) * TPW
    row0 = wid * TPW
    qsems = [semQ1, semQ2, semQ3]

    first = [
        pltpu.async_copy(pos_hbm.at[pl.ds(col0, QROWS)],
                         p_v.at[pl.ds(0, QROWS)], semA),
        pltpu.async_copy(tt_hbm.at[b, pl.ds(col0, TPW)], tt_v, semA),
        pltpu.async_copy(wm_hbm.at[b, pl.ds(col0, TPW)], wm_v, semA),
        pltpu.async_copy(ty_hbm, ty_v, semA),
        pltpu.async_copy(ar_hbm, ar_v, semA),
    ]
    pltpu.sync_copy(ids_hbm.at[b, pl.ds(col0, TPW)], idx_v)
    first.append(
        pltpu.async_copy(word_hbm.at[idx_v.at[pl.ds(0, QROWS)]],
                         w_v.at[pl.ds(0, QROWS)], semA))
    for q in range(1, NQ):
        sem = qsems[q - 1]
        pltpu.async_copy(word_hbm.at[idx_v.at[pl.ds(q * QROWS, QROWS)]],
                         w_v.at[pl.ds(q * QROWS, QROWS)], sem)
        pltpu.async_copy(pos_hbm.at[pl.ds(col0 + q * QROWS, QROWS)],
                         p_v.at[pl.ds(q * QROWS, QROWS)], sem)
    for cp in first:
        cp.wait()

    base_c = []
    dt_c = []
    da_c = []
    for c in range(CHUNKS):
        sl = pl.ds(c * L, L)
        t0 = ty_v[0, sl]
        a0 = ar_v[0, sl]
        base_c.append(t0 + a0)
        dt_c.append(ty_v[1, sl] - t0)
        da_c.append(ar_v[1, sl] - a0)

    def group_body(g, carry):
        for q in range(1, NQ):
            @pl.when(g == q * QGROUPS)
            def _(q=q):
                pltpu.make_async_copy(
                    word_hbm.at[pl.ds(0, QROWS)],
                    w_v.at[pl.ds(q * QROWS, QROWS)], qsems[q - 1]).wait()
                pltpu.make_async_copy(
                    pos_hbm.at[pl.ds(0, QROWS)],
                    p_v.at[pl.ds(q * QROWS, QROWS)], qsems[q - 1]).wait()
                pltpu.async_copy(
                    w_v.at[pl.ds((q - 1) * QROWS, QROWS)],
                    out_hbm.at[pl.ds(row0 + (q - 1) * QROWS, QROWS)],
                    semC)

        base = pl.multiple_of(g * L, L)
        ttf = tt_v[pl.ds(base, L)].astype(jnp.float32)
        wmf = wm_v[pl.ds(base, L)].astype(jnp.float32)

        for j in range(L):
            t = base + j
            ttj = _lane_bcast(ttf, j)
            wmj = _lane_bcast(wmf, j)
            for c in range(CHUNKS):
                sl = pl.ds(c * L, L)
                x = w_v[t, sl] + p_v[t, sl] + base_c[c]
                w_v[t, sl] = x + ttj * dt_c[c] + wmj * da_c[c]
        return carry

    lax.fori_loop(0, GROUPS, group_body, 0)

    pltpu.async_copy(w_v.at[pl.ds((NQ - 1) * QROWS, QROWS)],
                     out_hbm.at[pl.ds(row0 + (NQ - 1) * QROWS, QROWS)],
                     semC)
    for _ in range(NQ):
        pltpu.make_async_copy(word_hbm.at[pl.ds(0, QROWS)],
                              w_v.at[pl.ds(0, QROWS)], semC).wait()


_BLK = 512


def _ln_body(x_ref, g_ref, b_ref, o_ref):
    x = x_ref[...]
    ones = jnp.ones((HIDDEN, 1), jnp.float32)
    s = jax.lax.dot_general(x, ones, (((1,), (0,)), ((), ())),
                            preferred_element_type=jnp.float32)
    q = jax.lax.dot_general(x * x, ones, (((1,), (0,)), ((), ())),
                            preferred_element_type=jnp.float32)
    mean = s * (1.0 / HIDDEN)
    var = q * (1.0 / HIDDEN) - mean * mean
    o_ref[...] = (x - mean) * lax.rsqrt(var + EPS) * g_ref[...] + b_ref[...]


_ln_call = pl.pallas_call(
    _ln_body,
    out_shape=jax.ShapeDtypeStruct((NTOK, HIDDEN), jnp.float32),
    grid=(NTOK // _BLK,),
    in_specs=[
        pl.BlockSpec((_BLK, HIDDEN), lambda i: (i, 0)),
        pl.BlockSpec((1, HIDDEN), lambda i: (0, 0)),
        pl.BlockSpec((1, HIDDEN), lambda i: (0, 0)),
    ],
    out_specs=pl.BlockSpec((_BLK, HIDDEN), lambda i: (i, 0)),
)


def kernel(input_ids, token_type_ids, word_mask, word_emb, pos_emb,
           type_emb, arr_emb, gamma, beta):
    x = _emb_sum_kernel(input_ids.astype(jnp.int32),
                        token_type_ids.astype(jnp.int32),
                        word_mask.astype(jnp.int32),
                        word_emb, pos_emb, type_emb, arr_emb)
    out = _ln_call(x, gamma[None, :], beta[None, :])
    return out.reshape(BATCH, SEQ, HIDDEN)


# speedup vs baseline: 1.1934x; 1.1934x over previous
import functools

import jax
import jax.numpy as jnp
from jax import lax
from jax.experimental import pallas as pl
from jax.experimental.pallas import tpu as pltpu
from jax.experimental.pallas import tpu_sc as plsc

VOCAB = 100000
HIDDEN = 128
SEQ = 2048
BATCH = 4
EPS = 1e-12
L = 16
NC = 2
NS = 16
NW = NC * NS
NTOK = BATCH * SEQ
TPW = NTOK // NW
WPB = SEQ // TPW
GROUPS = TPW // L
CHUNKS = HIDDEN // L
NQ = 4
QROWS = TPW // NQ
QGROUPS = GROUPS // NQ


_DN = lax.GatherDimensionNumbers(
    offset_dims=(), collapsed_slice_dims=(0,), start_index_map=(0,))


def _gather16(v, idx):
    return lax.gather(v, idx[:, None], _DN, slice_sizes=(1,),
                      mode=lax.GatherScatterMode.PROMISE_IN_BOUNDS)


def _lane_bcast(v, j):
    return _gather16(v, jnp.full((L,), j, dtype=jnp.int32))


def _allsum(v):
    iota = jnp.arange(L, dtype=jnp.int32)
    for k in (1, 2, 4, 8):
        v = v + _gather16(v, iota ^ k)
    return v


_MESH = plsc.VectorSubcoreMesh(core_axis_name="c", subcore_axis_name="s")


@functools.partial(
    pl.kernel,
    out_type=jax.ShapeDtypeStruct((BATCH, SEQ, HIDDEN), jnp.float32),
    mesh=_MESH,
    scratch_types=[
        pltpu.VMEM((TPW,), jnp.int32),
        pltpu.VMEM((TPW, HIDDEN), jnp.float32),
        pltpu.VMEM((TPW, HIDDEN), jnp.float32),
        pltpu.VMEM((TPW,), jnp.int32),
        pltpu.VMEM((TPW,), jnp.int32),
        pltpu.VMEM((2, HIDDEN), jnp.float32),
        pltpu.VMEM((2, HIDDEN), jnp.float32),
        pltpu.VMEM((2, HIDDEN), jnp.float32),
        pltpu.SemaphoreType.DMA,
        pltpu.SemaphoreType.DMA,
        pltpu.SemaphoreType.DMA,
        pltpu.SemaphoreType.DMA,
        pltpu.SemaphoreType.DMA,
    ],
)
def _emb_ln_kernel(ids_hbm, tt_hbm, wm_hbm, word_hbm, pos_hbm, ty_hbm,
                   ar_hbm, gamma_hbm, beta_hbm, out_hbm,
                   idx_v, w_v, p_v, tt_v, wm_v, ty_v, ar_v, gb_v,
                   semA, semQ1, semQ2, semQ3, semC):
    cid = lax.axis_index("c")
    sid = lax.axis_index("s")
    wid = sid * NC + cid
    b = wid // WPB
    col0 = (wid % WPB) * TPW
    qsems = [semQ1, semQ2, semQ3]

    first = [
        pltpu.async_copy(pos_hbm.at[pl.ds(col0, QROWS)],
                         p_v.at[pl.ds(0, QROWS)], semA),
        pltpu.async_copy(tt_hbm.at[b, pl.ds(col0, TPW)], tt_v, semA),
        pltpu.async_copy(wm_hbm.at[b, pl.ds(col0, TPW)], wm_v, semA),
        pltpu.async_copy(ty_hbm, ty_v, semA),
        pltpu.async_copy(ar_hbm, ar_v, semA),
        pltpu.async_copy(gamma_hbm, gb_v.at[0], semA),
        pltpu.async_copy(beta_hbm, gb_v.at[1], semA),
    ]
    pltpu.sync_copy(ids_hbm.at[b, pl.ds(col0, TPW)], idx_v)
    first.append(
        pltpu.async_copy(word_hbm.at[idx_v.at[pl.ds(0, QROWS)]],
                         w_v.at[pl.ds(0, QROWS)], semA))
    for q in range(1, NQ):
        sem = qsems[q - 1]
        pltpu.async_copy(word_hbm.at[idx_v.at[pl.ds(q * QROWS, QROWS)]],
                         w_v.at[pl.ds(q * QROWS, QROWS)], sem)
        pltpu.async_copy(pos_hbm.at[pl.ds(col0 + q * QROWS, QROWS)],
                         p_v.at[pl.ds(q * QROWS, QROWS)], sem)
    for cp in first:
        cp.wait()

    base_c = []
    dt_c = []
    da_c = []
    g_c = []
    b_c = []
    for c in range(CHUNKS):
        sl = pl.ds(c * L, L)
        t0 = ty_v[0, sl]
        a0 = ar_v[0, sl]
        base_c.append(t0 + a0)
        dt_c.append(ty_v[1, sl] - t0)
        da_c.append(ar_v[1, sl] - a0)
        g_c.append(gb_v[0, sl])
        b_c.append(gb_v[1, sl])

    def group_body(g, carry):
        for q in range(1, NQ):
            @pl.when(g == q * QGROUPS)
            def _(q=q):
                pltpu.make_async_copy(
                    word_hbm.at[pl.ds(0, QROWS)],
                    w_v.at[pl.ds(q * QROWS, QROWS)], qsems[q - 1]).wait()
                pltpu.make_async_copy(
                    pos_hbm.at[pl.ds(0, QROWS)],
                    p_v.at[pl.ds(q * QROWS, QROWS)], qsems[q - 1]).wait()
                pltpu.async_copy(
                    w_v.at[pl.ds((q - 1) * QROWS, QROWS)],
                    out_hbm.at[b, pl.ds(col0 + (q - 1) * QROWS, QROWS)],
                    semC)

        base = pl.multiple_of(g * L, L)
        ttf = tt_v[pl.ds(base, L)].astype(jnp.float32)
        wmf = wm_v[pl.ds(base, L)].astype(jnp.float32)

        for j in range(L):
            t = base + j
            ttj = _lane_bcast(ttf, j)
            wmj = _lane_bcast(wmf, j)
            sv = None
            qv = None
            for c in range(CHUNKS):
                sl = pl.ds(c * L, L)
                x = w_v[t, sl] + p_v[t, sl] + base_c[c]
                x = x + ttj * dt_c[c] + wmj * da_c[c]
                w_v[t, sl] = x
                sv = x if sv is None else sv + x
                qv = x * x if qv is None else qv + x * x
            mean = _allsum(sv) * (1.0 / HIDDEN)
            q_all = _allsum(qv)
            var = q_all * (1.0 / HIDDEN) - mean * mean
            a = var + EPS
            ai = lax.bitcast_convert_type(a, jnp.int32)
            y = lax.bitcast_convert_type(
                jnp.int32(0x5F3759DF) - lax.shift_right_arithmetic(ai, 1),
                jnp.float32)
            ah = a * 0.5
            for _ in range(2):
                y = y * (1.5 - ah * y * y)
            for c in range(CHUNKS):
                sl = pl.ds(c * L, L)
                w_v[t, sl] = (w_v[t, sl] - mean) * y * g_c[c] + b_c[c]
        return carry

    lax.fori_loop(0, GROUPS, group_body, 0)

    pltpu.async_copy(w_v.at[pl.ds((NQ - 1) * QROWS, QROWS)],
                     out_hbm.at[b, pl.ds(col0 + (NQ - 1) * QROWS, QROWS)],
                     semC)
    for _ in range(NQ):
        pltpu.make_async_copy(word_hbm.at[pl.ds(0, QROWS)],
                              w_v.at[pl.ds(0, QROWS)], semC).wait()


def kernel(input_ids, token_type_ids, word_mask, word_emb, pos_emb,
           type_emb, arr_emb, gamma, beta):
    return _emb_ln_kernel(input_ids.astype(jnp.int32),
                          token_type_ids.astype(jnp.int32),
                          word_mask.astype(jnp.int32),
                          word_emb, pos_emb, type_emb, arr_emb, gamma, beta)


# speedup vs baseline: 1.2516x; 1.0488x over previous
import functools

import jax
import jax.numpy as jnp
from jax import lax
from jax.experimental import pallas as pl
from jax.experimental.pallas import tpu as pltpu
from jax.experimental.pallas import tpu_sc as plsc

VOCAB = 100000
HIDDEN = 128
SEQ = 2048
BATCH = 4
EPS = 1e-12
L = 16
NC = 2
NS = 16
NW = NC * NS
NTOK = BATCH * SEQ
TPW = NTOK // NW
WPB = SEQ // TPW
GROUPS = TPW // L
CHUNKS = HIDDEN // L
NQ = 4
QROWS = TPW // NQ
QGROUPS = GROUPS // NQ


_DN = lax.GatherDimensionNumbers(
    offset_dims=(), collapsed_slice_dims=(0,), start_index_map=(0,))


def _gather16(v, idx):
    return lax.gather(v, idx[:, None], _DN, slice_sizes=(1,),
                      mode=lax.GatherScatterMode.PROMISE_IN_BOUNDS)


def _lane_bcast(v, j):
    return _gather16(v, jnp.full((L,), j, dtype=jnp.int32))


def _allsum(v):
    iota = jnp.arange(L, dtype=jnp.int32)
    for k in (1, 2, 4, 8):
        v = v + _gather16(v, iota ^ k)
    return v


_MESH = plsc.VectorSubcoreMesh(core_axis_name="c", subcore_axis_name="s")


@functools.partial(
    pl.kernel,
    out_type=jax.ShapeDtypeStruct((BATCH, SEQ, HIDDEN), jnp.float32),
    mesh=_MESH,
    scratch_types=[
        pltpu.VMEM((TPW,), jnp.int32),
        pltpu.VMEM((TPW, HIDDEN), jnp.float32),
        pltpu.VMEM((TPW, HIDDEN), jnp.float32),
        pltpu.VMEM((TPW,), jnp.int32),
        pltpu.VMEM((TPW,), jnp.int32),
        pltpu.VMEM((2, HIDDEN), jnp.float32),
        pltpu.VMEM((2, HIDDEN), jnp.float32),
        pltpu.VMEM((2, HIDDEN), jnp.float32),
        pltpu.SemaphoreType.DMA,
        pltpu.SemaphoreType.DMA,
        pltpu.SemaphoreType.DMA,
        pltpu.SemaphoreType.DMA,
        pltpu.SemaphoreType.DMA,
    ],
)
def _emb_ln_kernel(ids_hbm, tt_hbm, wm_hbm, word_hbm, pos_hbm, ty_hbm,
                   ar_hbm, gamma_hbm, beta_hbm, out_hbm,
                   idx_v, w_v, p_v, tt_v, wm_v, ty_v, ar_v, gb_v,
                   semA, semQ1, semQ2, semQ3, semC):
    cid = lax.axis_index("c")
    sid = lax.axis_index("s")
    wid = sid * NC + cid
    b = wid // WPB
    col0 = (wid % WPB) * TPW
    qsems = [semQ1, semQ2, semQ3]

    first = [
        pltpu.async_copy(pos_hbm.at[pl.ds(col0, QROWS)],
                         p_v.at[pl.ds(0, QROWS)], semA),
        pltpu.async_copy(tt_hbm.at[b, pl.ds(col0, TPW)], tt_v, semA),
        pltpu.async_copy(wm_hbm.at[b, pl.ds(col0, TPW)], wm_v, semA),
        pltpu.async_copy(ty_hbm, ty_v, semA),
        pltpu.async_copy(ar_hbm, ar_v, semA),
        pltpu.async_copy(gamma_hbm, gb_v.at[0], semA),
        pltpu.async_copy(beta_hbm, gb_v.at[1], semA),
    ]
    pltpu.sync_copy(ids_hbm.at[b, pl.ds(col0, TPW)], idx_v)
    first.append(
        pltpu.async_copy(word_hbm.at[idx_v.at[pl.ds(0, QROWS)]],
                         w_v.at[pl.ds(0, QROWS)], semA))
    for q in range(1, NQ):
        sem = qsems[q - 1]
        pltpu.async_copy(word_hbm.at[idx_v.at[pl.ds(q * QROWS, QROWS)]],
                         w_v.at[pl.ds(q * QROWS, QROWS)], sem)
        pltpu.async_copy(pos_hbm.at[pl.ds(col0 + q * QROWS, QROWS)],
                         p_v.at[pl.ds(q * QROWS, QROWS)], sem)
    for cp in first:
        cp.wait()

    base_c = []
    dt_c = []
    da_c = []
    g_c = []
    b_c = []
    for c in range(CHUNKS):
        sl = pl.ds(c * L, L)
        t0 = ty_v[0, sl]
        a0 = ar_v[0, sl]
        base_c.append(t0 + a0)
        dt_c.append(ty_v[1, sl] - t0)
        da_c.append(ar_v[1, sl] - a0)
        g_c.append(gb_v[0, sl])
        b_c.append(gb_v[1, sl])

    def group_body(g, carry):
        for q in range(1, NQ):
            @pl.when(g == q * QGROUPS)
            def _(q=q):
                pltpu.make_async_copy(
                    word_hbm.at[pl.ds(0, QROWS)],
                    w_v.at[pl.ds(q * QROWS, QROWS)], qsems[q - 1]).wait()
                pltpu.make_async_copy(
                    pos_hbm.at[pl.ds(0, QROWS)],
                    p_v.at[pl.ds(q * QROWS, QROWS)], qsems[q - 1]).wait()
                pltpu.async_copy(
                    w_v.at[pl.ds((q - 1) * QROWS, QROWS)],
                    out_hbm.at[b, pl.ds(col0 + (q - 1) * QROWS, QROWS)],
                    semC)

        base = pl.multiple_of(g * L, L)
        ttf = tt_v[pl.ds(base, L)].astype(jnp.float32)
        wmf = wm_v[pl.ds(base, L)].astype(jnp.float32)
        iota = jnp.arange(L, dtype=jnp.int32)

        sg = jnp.zeros((L,), jnp.float32)
        qg = jnp.zeros((L,), jnp.float32)
        for j in range(L):
            t = base + j
            ttj = _lane_bcast(ttf, j)
            wmj = _lane_bcast(wmf, j)
            sv = None
            qv = None
            for c in range(CHUNKS):
                sl = pl.ds(c * L, L)
                x = w_v[t, sl] + p_v[t, sl] + base_c[c]
                x = x + ttj * dt_c[c] + wmj * da_c[c]
                w_v[t, sl] = x
                sv = x if sv is None else sv + x
                qv = x * x if qv is None else qv + x * x
            msk = iota == j
            sg = jnp.where(msk, _allsum(sv), sg)
            qg = jnp.where(msk, _allsum(qv), qg)

        mean = sg * (1.0 / HIDDEN)
        var = qg * (1.0 / HIDDEN) - mean * mean
        a = var + EPS
        ai = lax.bitcast_convert_type(a, jnp.int32)
        y = lax.bitcast_convert_type(
            jnp.int32(0x5F3759DF) - lax.shift_right_arithmetic(ai, 1),
            jnp.float32)
        ah = a * 0.5
        for _ in range(2):
            y = y * (1.5 - ah * y * y)

        for j in range(L):
            t = base + j
            mj = _lane_bcast(mean, j)
            yj = _lane_bcast(y, j)
            for c in range(CHUNKS):
                sl = pl.ds(c * L, L)
                w_v[t, sl] = (w_v[t, sl] - mj) * yj * g_c[c] + b_c[c]
        return carry

    lax.fori_loop(0, GROUPS, group_body, 0)

    pltpu.async_copy(w_v.at[pl.ds((NQ - 1) * QROWS, QROWS)],
                     out_hbm.at[b, pl.ds(col0 + (NQ - 1) * QROWS, QROWS)],
                     semC)
    for _ in range(NQ):
        pltpu.make_async_copy(word_hbm.at[pl.ds(0, QROWS)],
                              w_v.at[pl.ds(0, QROWS)], semC).wait()


def kernel(input_ids, token_type_ids, word_mask, word_emb, pos_emb,
           type_emb, arr_emb, gamma, beta):
    return _emb_ln_kernel(input_ids.astype(jnp.int32),
                          token_type_ids.astype(jnp.int32),
                          word_mask.astype(jnp.int32),
                          word_emb, pos_emb, type_emb, arr_emb, gamma, beta)
